# baseline (speedup 1.0000x reference)
import functools

import jax
import jax.numpy as jnp
from jax import lax
from jax.experimental import pallas as pl
from jax.experimental.pallas import tpu as pltpu
from jax.experimental.pallas import tpu_sc as plsc

_ROWS = 16384
_COLS = 50
_V = 1000000
_D = 64
_DP = 128
_NC, _NS = 2, 16
_NW = _NC * _NS
_IB = 128
_IPW = _ROWS // _NW
_IBW = _IPW // _IB
_NCH = _COLS * _IBW
_NBUF = 4

_TA = 4096


def _fmt_table_body(t_ref, o_ref):
    o_ref[:, :_D] = jnp.swapaxes(t_ref[...], 0, 1)
    o_ref[:, _D:] = jnp.zeros((_TA, _DP - _D), jnp.float32)


def _fmt_table(tT):
    return pl.pallas_call(
        _fmt_table_body,
        grid=((_V + _TA - 1) // _TA,),
        in_specs=[pl.BlockSpec((_D, _TA), lambda i: (0, i))],
        out_specs=pl.BlockSpec((_TA, _DP), lambda i: (i, 0)),
        out_shape=jax.ShapeDtypeStruct((_V, _DP), jnp.float32),
    )(tT)


_mesh = plsc.VectorSubcoreMesh(core_axis_name="c", subcore_axis_name="s")


@functools.partial(
    pl.kernel,
    out_type=jax.ShapeDtypeStruct((_COLS, _ROWS, _DP), jnp.float32),
    mesh=_mesh,
    scratch_types=[
        pltpu.VMEM((_COLS, _IPW), jnp.int32),
        pltpu.VMEM((_NBUF, _IB, _DP), jnp.float32),
        [pltpu.SemaphoreType.DMA] * _NBUF,
        [pltpu.SemaphoreType.DMA] * _NBUF,
    ],
    compiler_params=pltpu.CompilerParams(use_tc_tiling_on_sc=False),
)
def _sc_gather(xT_hbm, tpad_hbm, g_hbm, idx_v, rows_v, gsems, fsems):
    wid = lax.axis_index("s") * _NC + lax.axis_index("c")
    ibase = wid * _IPW

    pltpu.sync_copy(xT_hbm.at[:, pl.ds(ibase, _IPW)], idx_v)

    def gather_copy(c, b):
        j = c // _IBW
        ib = c % _IBW
        return pltpu.make_async_copy(
            tpad_hbm.at[idx_v.at[j, pl.ds(ib * _IB, _IB)]],
            rows_v.at[b],
            gsems[b],
        )

    def flush_copy(c, b):
        j = c // _IBW
        ib = c % _IBW
        return pltpu.make_async_copy(
            rows_v.at[b],
            g_hbm.at[j, pl.ds(ibase + ib * _IB, _IB)],
            fsems[b],
        )

    for b in range(_NBUF):
        gather_copy(b, b).start()

    @pl.loop(0, _NCH, step=_NBUF)
    def _(c):
        for b in range(_NBUF):
            gather_copy(c + b, b).wait()
            flush_copy(c + b, b).start()
        for b in range(_NBUF):
            flush_copy(c + b, b).wait()

            @pl.when(c + _NBUF + b < _NCH)
            def _():
                gather_copy(c + _NBUF + b, b).start()


def _fmt_out_body(g_ref, o_ref):
    o_ref[0] = jnp.swapaxes(g_ref[0, :, :_D], 0, 1)


def _fmt_out(g):
    return pl.pallas_call(
        _fmt_out_body,
        grid=(_COLS, _ROWS // _IB),
        in_specs=[pl.BlockSpec((1, _IB, _DP), lambda j, i: (j, i, 0))],
        out_specs=pl.BlockSpec((1, _D, _IB), lambda j, i: (j, 0, i)),
        out_shape=jax.ShapeDtypeStruct((_COLS, _D, _ROWS), jnp.float32),
    )(g)


def kernel(x, table):
    tT = jnp.swapaxes(table, 0, 1)
    tpad = _fmt_table(tT)
    xT = jnp.swapaxes(x, 0, 1)
    g = _sc_gather(xT, tpad)
    outT = _fmt_out(g)
    return outT.transpose(2, 0, 1)


# speedup vs baseline: 5.6937x; 5.6937x over previous
import functools

import jax
import jax.numpy as jnp
from jax import lax
from jax.experimental import pallas as pl
from jax.experimental.pallas import tpu as pltpu
from jax.experimental.pallas import tpu_sc as plsc

_ROWS = 16384
_COLS = 50
_V = 1000000
_D = 64
_DP = 128
_NC, _NS = 2, 16
_NW = _NC * _NS
_IB = 128
_IPW = _ROWS // _NW
_IBW = _IPW // _IB
_NCH = _COLS * _IBW
_NBUF = 4

_TA = 4096


def _fmt_table_body(t_ref, o_ref):
    o_ref[:, :_D] = jnp.swapaxes(t_ref[...], 0, 1)


def _fmt_table(tT):
    return pl.pallas_call(
        _fmt_table_body,
        grid=((_V + _TA - 1) // _TA,),
        in_specs=[pl.BlockSpec((_D, _TA), lambda i: (0, i))],
        out_specs=pl.BlockSpec((_TA, _DP), lambda i: (i, 0)),
        out_shape=jax.ShapeDtypeStruct((_V, _DP), jnp.float32),
    )(tT)


_mesh = plsc.VectorSubcoreMesh(core_axis_name="c", subcore_axis_name="s")


@functools.partial(
    pl.kernel,
    out_type=jax.ShapeDtypeStruct((_COLS, _ROWS, _DP), jnp.float32),
    mesh=_mesh,
    scratch_types=[
        pltpu.VMEM((_COLS, _IPW), jnp.int32),
        pltpu.VMEM((_NBUF, _IB, _D), jnp.float32),
        [pltpu.SemaphoreType.DMA] * _NBUF,
        [pltpu.SemaphoreType.DMA] * _NBUF,
    ],
    compiler_params=pltpu.CompilerParams(use_tc_tiling_on_sc=False),
)
def _sc_gather(x2_hbm, tab2_hbm, g_hbm, idx_v, rows_v, gsems, fsems):
    wid = lax.axis_index("s") * _NC + lax.axis_index("c")
    ibase = wid * _IPW

    pltpu.sync_copy(x2_hbm.at[:, pl.ds(ibase, _IPW)], idx_v)

    def gather_copy(c, b):
        j = c // _IBW
        ib = c % _IBW
        return pltpu.make_async_copy(
            tab2_hbm.at[idx_v.at[j, pl.ds(ib * _IB, _IB)]],
            rows_v.at[b],
            gsems[b],
        )

    def flush_copy(c, b):
        j = c // _IBW
        ib = c % _IBW
        return pltpu.make_async_copy(
            rows_v.at[b],
            g_hbm.at[j, pl.ds(ibase + ib * _IB, _IB), pl.ds(0, _D)],
            fsems[b],
        )

    for b in range(_NBUF):
        gather_copy(b, b).start()

    @pl.loop(0, _NCH, step=_NBUF)
    def _(c):
        for b in range(_NBUF):
            gather_copy(c + b, b).wait()
            flush_copy(c + b, b).start()
        for b in range(_NBUF):
            flush_copy(c + b, b).wait()

            @pl.when(c + _NBUF + b < _NCH)
            def _():
                gather_copy(c + _NBUF + b, b).start()


def _fmt_out_body(g_ref, o_ref):
    o_ref[0] = jnp.swapaxes(g_ref[0, :, :_D], 0, 1)


def _fmt_out(g):
    return pl.pallas_call(
        _fmt_out_body,
        grid=(_COLS,),
        in_specs=[pl.BlockSpec((1, _ROWS, _DP), lambda j: (j, 0, 0))],
        out_specs=pl.BlockSpec((1, _D, _ROWS), lambda j: (j, 0, 0)),
        out_shape=jax.ShapeDtypeStruct((_COLS, _D, _ROWS), jnp.float32),
        compiler_params=pltpu.CompilerParams(vmem_limit_bytes=100 * 1024 * 1024),
    )(g)


def kernel(x, table):
    tT = jnp.swapaxes(table, 0, 1)
    tpad = _fmt_table(tT)
    tab2 = tpad.reshape(2 * _V, _D)
    x2 = jnp.swapaxes(x, 0, 1) * 2
    g = _sc_gather(x2, tab2)
    outT = _fmt_out(g)
    return outT.transpose(2, 0, 1)
